# baseline (speedup 1.0000x reference)
import functools

import jax
import jax.numpy as jnp
from jax import lax
from jax.experimental import pallas as pl
from jax.experimental.pallas import tpu as pltpu
from jax.experimental.pallas import tpu_sc as plsc

N = 10000
E = 320000
D = 128
FP = 512
R = 3
EPS = 1e-5

NC = 2
NS = 16
NW = NC * NS
CH = 128
NCHUNK = 80
EPT = CH * NCHUNK
E_PAD = EPT * NW
ACC_ROWS = 10112
TROWS = ACC_ROWS - N
APT = ACC_ROWS // NS
ZROWS = 32
GRP = 16
NPAIR = NCHUNK // 2

BN_BLK = 2000
NB = N // BN_BLK


def _sc_agg_body(feats_hbm, src_hbm, dst_hbm, out_hbm,
                 srcv, dstv, rows_a, rows_b, zbuf, acc,
                 sem_a, sem_b, sem_z, sem_sa, sem_sb):
  c = lax.axis_index("c")
  s = lax.axis_index("s")
  wid = s * NC + c

  @pl.loop(0, ZROWS)
  def _(r):
    @pl.loop(0, D, step=16)
    def _(l):
      zbuf[r, pl.ds(l, 16)] = jnp.zeros((16,), jnp.float32)

  zbase = pl.multiple_of(s * APT, 8)

  @pl.loop(0, 19)
  def _(k):
    off = pl.multiple_of(zbase + k * ZROWS, 8)
    pltpu.async_copy(zbuf, acc.at[pl.ds(off, ZROWS)], sem_z)

  pltpu.async_copy(zbuf.at[pl.ds(0, 24)],
                   acc.at[pl.ds(pl.multiple_of(zbase + 608, 8), 24)], sem_z)
  pltpu.async_copy(src_hbm.at[wid], srcv, sem_b)

  pltpu.make_async_copy(src_hbm.at[wid], srcv, sem_b).wait()
  pltpu.async_copy(feats_hbm.at[srcv.at[0]], rows_a, sem_a)

  @pl.loop(0, 19)
  def _(k):
    off = pl.multiple_of(zbase + k * ZROWS, 8)
    pltpu.make_async_copy(zbuf, acc.at[pl.ds(off, ZROWS)], sem_z).wait()

  pltpu.make_async_copy(
      zbuf.at[pl.ds(0, 24)],
      acc.at[pl.ds(pl.multiple_of(zbase + 608, 8), 24)], sem_z).wait()

  plsc.subcore_barrier()

  @pl.loop(0, NPAIR)
  def _(p):
    k0 = 2 * p
    r0 = k0 % GRP

    @pl.when(p > 0)
    def _():
      pltpu.make_async_copy(rows_b, acc.at[dstv.at[GRP - 1]], sem_sb).wait()

    pltpu.async_copy(feats_hbm.at[srcv.at[k0 + 1]], rows_b, sem_b)

    @pl.when(k0 % GRP == 0)
    def _():
      goff = pl.multiple_of(k0, 8)
      pltpu.sync_copy(dst_hbm.at[wid].at[pl.ds(goff, GRP)], dstv)

    pltpu.make_async_copy(feats_hbm.at[srcv.at[k0]], rows_a, sem_a).wait()
    sca = pltpu.async_copy(rows_a, acc.at[dstv.at[r0]], sem_sa, add=True)

    pltpu.make_async_copy(feats_hbm.at[srcv.at[k0 + 1]], rows_b, sem_b).wait()
    sca.wait()

    @pl.when(p < NPAIR - 1)
    def _():
      pltpu.async_copy(feats_hbm.at[srcv.at[k0 + 2]], rows_a, sem_a)

    pltpu.async_copy(rows_b, acc.at[dstv.at[r0 + 1]], sem_sb, add=True)

  pltpu.make_async_copy(rows_b, acc.at[dstv.at[GRP - 1]], sem_sb).wait()

  plsc.subcore_barrier()

  @pl.when(s < NS - 1)
  def _():
    base = pl.multiple_of(s * 624, 8)
    pltpu.sync_copy(acc.at[pl.ds(base, 624)],
                    out_hbm.at[c].at[pl.ds(base, 624)])

  @pl.when(s == NS - 1)
  def _():
    base = (NS - 1) * 624
    pltpu.sync_copy(acc.at[pl.ds(base, 640)],
                    out_hbm.at[c].at[pl.ds(base, 640)])


@jax.jit
def _sc_aggregate(feats, src3, dst3):
  mesh = plsc.VectorSubcoreMesh(core_axis_name="c", subcore_axis_name="s")
  kern = pl.kernel(
      _sc_agg_body,
      out_type=jax.ShapeDtypeStruct((NC, N, D), jnp.float32),
      mesh=mesh,
      scratch_types=[
          pltpu.VMEM((NCHUNK, CH), jnp.int32),
          pltpu.VMEM((GRP, CH), jnp.int32),
          pltpu.VMEM((CH, D), jnp.float32),
          pltpu.VMEM((CH, D), jnp.float32),
          pltpu.VMEM((ZROWS, D), jnp.float32),
          pltpu.VMEM_SHARED((ACC_ROWS, D), jnp.float32),
          pltpu.SemaphoreType.DMA,
          pltpu.SemaphoreType.DMA,
          pltpu.SemaphoreType.DMA,
          pltpu.SemaphoreType.DMA,
          pltpu.SemaphoreType.DMA,
      ],
  )
  return kern(feats, src3, dst3)


def _fp_body(f_ref, w_ref, b_ref, o_ref):
  i = pl.program_id(0)
  z = lax.dot_general(f_ref[...], w_ref[...],
                      dimension_numbers=(((1,), (1,)), ((), ())),
                      preferred_element_type=jnp.float32,
                      precision=lax.Precision.HIGHEST)
  z = z + b_ref[...]
  m = jnp.max(z, axis=1, keepdims=True)
  e = jnp.exp(z - m)
  p = e / jnp.sum(e, axis=1, keepdims=True)
  blk = jnp.sum(p, axis=0, keepdims=True)

  @pl.when(i == 0)
  def _():
    o_ref[...] = jnp.zeros_like(o_ref)

  o_ref[...] += blk


@jax.jit
def _fp_contrib(feats, w, b2d):
  return pl.pallas_call(
      _fp_body,
      grid=(NB,),
      in_specs=[
          pl.BlockSpec((BN_BLK, D), lambda i: (i, 0)),
          pl.BlockSpec((FP, D), lambda i: (0, 0)),
          pl.BlockSpec((1, FP), lambda i: (0, 0)),
      ],
      out_specs=pl.BlockSpec((1, FP), lambda i: (0, 0)),
      out_shape=jax.ShapeDtypeStruct((1, FP), jnp.float32),
  )(feats, w, b2d)


def _bn_body(f_ref, p_ref, wh_ref, bh_ref, g_ref, bt_ref, hn_ref,
             h_scr, st_scr):
  i = pl.program_id(0)

  @pl.when(i < NB)
  def _():
    agg = f_ref[...] + p_ref[0] + p_ref[1]
    h = lax.dot_general(agg, wh_ref[...],
                        dimension_numbers=(((1,), (1,)), ((), ())),
                        preferred_element_type=jnp.float32,
                        precision=lax.Precision.HIGHEST)
    h = jnp.maximum(h + bh_ref[...], 0.0)
    h_scr[pl.ds(i * BN_BLK, BN_BLK), :] = h

    @pl.when(i == 0)
    def _():
      st_scr[...] = jnp.zeros_like(st_scr)

    st_scr[0:1, :] += jnp.sum(h, axis=0, keepdims=True)
    st_scr[1:2, :] += jnp.sum(h * h, axis=0, keepdims=True)

  @pl.when(i >= NB)
  def _():
    j = i - NB
    mean = st_scr[0:1, :] * (1.0 / N)
    var = st_scr[1:2, :] * (1.0 / N) - mean * mean
    rstd = lax.rsqrt(var + EPS)
    scale = g_ref[...] * rstd
    shift = bt_ref[...] - mean * scale
    h = h_scr[pl.ds(j * BN_BLK, BN_BLK), :]
    hn_ref[...] = h * scale + shift


@jax.jit
def _bn_layer(feats, partials, wh, bh2d, g2d, bt2d):
  return pl.pallas_call(
      _bn_body,
      grid=(2 * NB,),
      in_specs=[
          pl.BlockSpec((BN_BLK, D),
                       lambda i: (jnp.where(i < NB, i, NB - 1), 0)),
          pl.BlockSpec((NC, BN_BLK, D),
                       lambda i: (0, jnp.where(i < NB, i, NB - 1), 0)),
          pl.BlockSpec((D, D), lambda i: (0, 0)),
          pl.BlockSpec((1, D), lambda i: (0, 0)),
          pl.BlockSpec((1, D), lambda i: (0, 0)),
          pl.BlockSpec((1, D), lambda i: (0, 0)),
      ],
      out_specs=pl.BlockSpec((BN_BLK, D),
                             lambda i: (jnp.where(i < NB, 0, i - NB), 0)),
      out_shape=jax.ShapeDtypeStruct((N, D), jnp.float32),
      scratch_shapes=[
          pltpu.VMEM((N, D), jnp.float32),
          pltpu.VMEM((2, D), jnp.float32),
      ],
  )(feats, partials, wh, bh2d, g2d, bt2d)


def kernel(x, edge_index, W0, b0, Wh, bh, Ws, bs, gamma, beta):
  src = edge_index[0]
  dst = edge_index[1]

  pad = E_PAD - E
  pad_src = (jnp.arange(pad, dtype=jnp.int32) * 37) % N
  pad_dst = N + (jnp.arange(pad, dtype=jnp.int32) % TROWS)
  src3 = jnp.concatenate([src, pad_src]).reshape(NW, NCHUNK, CH)
  dst3 = jnp.concatenate([dst, pad_dst]).reshape(NW, NCHUNK, CH)

  b02 = b0.reshape(1, FP)
  g2d = gamma.reshape(1, D)
  bt2d = beta.reshape(1, D)

  fp = _fp_contrib(x, W0, b02)

  feats = x
  for l in range(R):
    partials = _sc_aggregate(feats, src3, dst3)
    hn = _bn_layer(feats, partials, Wh[l], bh[l].reshape(1, D), g2d, bt2d)
    fp = fp + _fp_contrib(hn, Ws[l], bs[l].reshape(1, FP))
    feats = hn

  return fp.reshape(1, FP)


# speedup vs baseline: 1.0264x; 1.0264x over previous
import functools

import jax
import jax.numpy as jnp
from jax import lax
from jax.experimental import pallas as pl
from jax.experimental.pallas import tpu as pltpu
from jax.experimental.pallas import tpu_sc as plsc

N = 10000
E = 320000
D = 128
FP = 512
R = 3
EPS = 1e-5

NC = 2
NS = 16
NW = NC * NS
CH = 128
NCHUNK = 80
EPT = CH * NCHUNK
E_PAD = EPT * NW
ACC_ROWS = 10112
TROWS = ACC_ROWS - N
APT = ACC_ROWS // NS
ZROWS = 32
GRP = 16
NPAIR = NCHUNK // 2

BN_BLK = 2000
NB = N // BN_BLK


def _sc_agg_body(feats_hbm, src_hbm, dst_hbm, out_hbm,
                 srcv, dstv, rows_a, rows_b, zbuf, acc, sem_a, sem_b, sem_z):
  c = lax.axis_index("c")
  s = lax.axis_index("s")
  wid = s * NC + c

  @pl.loop(0, ZROWS)
  def _(r):
    @pl.loop(0, D, step=16)
    def _(l):
      zbuf[r, pl.ds(l, 16)] = jnp.zeros((16,), jnp.float32)

  zbase = pl.multiple_of(s * APT, 8)

  @pl.loop(0, 19)
  def _(k):
    off = pl.multiple_of(zbase + k * ZROWS, 8)
    pltpu.async_copy(zbuf, acc.at[pl.ds(off, ZROWS)], sem_z)

  pltpu.async_copy(zbuf.at[pl.ds(0, 24)],
                   acc.at[pl.ds(pl.multiple_of(zbase + 608, 8), 24)], sem_z)
  pltpu.async_copy(src_hbm.at[wid], srcv, sem_b)

  pltpu.make_async_copy(src_hbm.at[wid], srcv, sem_b).wait()
  pltpu.async_copy(feats_hbm.at[srcv.at[0]], rows_a, sem_a)

  @pl.loop(0, 19)
  def _(k):
    off = pl.multiple_of(zbase + k * ZROWS, 8)
    pltpu.make_async_copy(zbuf, acc.at[pl.ds(off, ZROWS)], sem_z).wait()

  pltpu.make_async_copy(
      zbuf.at[pl.ds(0, 24)],
      acc.at[pl.ds(pl.multiple_of(zbase + 608, 8), 24)], sem_z).wait()

  plsc.subcore_barrier()

  @pl.loop(0, NPAIR)
  def _(p):
    k0 = 2 * p
    pltpu.async_copy(feats_hbm.at[srcv.at[k0 + 1]], rows_b, sem_b)

    @pl.when(k0 % GRP == 0)
    def _():
      goff = pl.multiple_of(k0, 8)
      pltpu.sync_copy(dst_hbm.at[wid].at[pl.ds(goff, GRP)], dstv)

    r0 = k0 % GRP
    pltpu.make_async_copy(feats_hbm.at[srcv.at[k0]], rows_a, sem_a).wait()
    pltpu.sync_copy(rows_a, acc.at[dstv.at[r0]], add=True)

    @pl.when(p < NPAIR - 1)
    def _():
      pltpu.async_copy(feats_hbm.at[srcv.at[k0 + 2]], rows_a, sem_a)

    pltpu.make_async_copy(feats_hbm.at[srcv.at[k0 + 1]], rows_b, sem_b).wait()
    pltpu.sync_copy(rows_b, acc.at[dstv.at[r0 + 1]], add=True)

  plsc.subcore_barrier()

  @pl.when(s < NS - 1)
  def _():
    base = pl.multiple_of(s * 624, 8)
    pltpu.sync_copy(acc.at[pl.ds(base, 624)],
                    out_hbm.at[c].at[pl.ds(base, 624)])

  @pl.when(s == NS - 1)
  def _():
    base = (NS - 1) * 624
    pltpu.sync_copy(acc.at[pl.ds(base, 640)],
                    out_hbm.at[c].at[pl.ds(base, 640)])


@jax.jit
def _sc_aggregate(feats, src3, dst3):
  mesh = plsc.VectorSubcoreMesh(core_axis_name="c", subcore_axis_name="s")
  kern = pl.kernel(
      _sc_agg_body,
      out_type=jax.ShapeDtypeStruct((NC, N, D), jnp.float32),
      mesh=mesh,
      scratch_types=[
          pltpu.VMEM((NCHUNK, CH), jnp.int32),
          pltpu.VMEM((GRP, CH), jnp.int32),
          pltpu.VMEM((CH, D), jnp.float32),
          pltpu.VMEM((CH, D), jnp.float32),
          pltpu.VMEM((ZROWS, D), jnp.float32),
          pltpu.VMEM_SHARED((ACC_ROWS, D), jnp.float32),
          pltpu.SemaphoreType.DMA,
          pltpu.SemaphoreType.DMA,
          pltpu.SemaphoreType.DMA,
      ],
  )
  return kern(feats, src3, dst3)


def _fp_body(f_ref, w_ref, b_ref, o_ref):
  i = pl.program_id(0)
  z = lax.dot_general(f_ref[...], w_ref[...],
                      dimension_numbers=(((1,), (1,)), ((), ())),
                      preferred_element_type=jnp.float32,
                      precision=lax.Precision.HIGHEST)
  z = z + b_ref[...]
  m = jnp.max(z, axis=1, keepdims=True)
  e = jnp.exp(z - m)
  p = e / jnp.sum(e, axis=1, keepdims=True)
  blk = jnp.sum(p, axis=0, keepdims=True)

  @pl.when(i == 0)
  def _():
    o_ref[...] = jnp.zeros_like(o_ref)

  o_ref[...] += blk


@jax.jit
def _fp_contrib(feats, w, b2d):
  return pl.pallas_call(
      _fp_body,
      grid=(NB,),
      in_specs=[
          pl.BlockSpec((BN_BLK, D), lambda i: (i, 0)),
          pl.BlockSpec((FP, D), lambda i: (0, 0)),
          pl.BlockSpec((1, FP), lambda i: (0, 0)),
      ],
      out_specs=pl.BlockSpec((1, FP), lambda i: (0, 0)),
      out_shape=jax.ShapeDtypeStruct((1, FP), jnp.float32),
  )(feats, w, b2d)


def _bn_body(f_ref, p_ref, wh_ref, bh_ref, g_ref, bt_ref, hn_ref,
             h_scr, st_scr):
  i = pl.program_id(0)

  @pl.when(i < NB)
  def _():
    agg = f_ref[...] + p_ref[0] + p_ref[1]
    h = lax.dot_general(agg, wh_ref[...],
                        dimension_numbers=(((1,), (1,)), ((), ())),
                        preferred_element_type=jnp.float32,
                        precision=lax.Precision.HIGHEST)
    h = jnp.maximum(h + bh_ref[...], 0.0)
    h_scr[pl.ds(i * BN_BLK, BN_BLK), :] = h

    @pl.when(i == 0)
    def _():
      st_scr[...] = jnp.zeros_like(st_scr)

    st_scr[0:1, :] += jnp.sum(h, axis=0, keepdims=True)
    st_scr[1:2, :] += jnp.sum(h * h, axis=0, keepdims=True)

  @pl.when(i >= NB)
  def _():
    j = i - NB
    mean = st_scr[0:1, :] * (1.0 / N)
    var = st_scr[1:2, :] * (1.0 / N) - mean * mean
    rstd = lax.rsqrt(var + EPS)
    scale = g_ref[...] * rstd
    shift = bt_ref[...] - mean * scale
    h = h_scr[pl.ds(j * BN_BLK, BN_BLK), :]
    hn_ref[...] = h * scale + shift


@jax.jit
def _bn_layer(feats, partials, wh, bh2d, g2d, bt2d):
  return pl.pallas_call(
      _bn_body,
      grid=(2 * NB,),
      in_specs=[
          pl.BlockSpec((BN_BLK, D),
                       lambda i: (jnp.where(i < NB, i, NB - 1), 0)),
          pl.BlockSpec((NC, BN_BLK, D),
                       lambda i: (0, jnp.where(i < NB, i, NB - 1), 0)),
          pl.BlockSpec((D, D), lambda i: (0, 0)),
          pl.BlockSpec((1, D), lambda i: (0, 0)),
          pl.BlockSpec((1, D), lambda i: (0, 0)),
          pl.BlockSpec((1, D), lambda i: (0, 0)),
      ],
      out_specs=pl.BlockSpec((BN_BLK, D),
                             lambda i: (jnp.where(i < NB, 0, i - NB), 0)),
      out_shape=jax.ShapeDtypeStruct((N, D), jnp.float32),
      scratch_shapes=[
          pltpu.VMEM((N, D), jnp.float32),
          pltpu.VMEM((2, D), jnp.float32),
      ],
  )(feats, partials, wh, bh2d, g2d, bt2d)


def kernel(x, edge_index, W0, b0, Wh, bh, Ws, bs, gamma, beta):
  src = edge_index[0]
  dst = edge_index[1]

  pad = E_PAD - E
  pad_src = (jnp.arange(pad, dtype=jnp.int32) * 37) % N
  pad_dst = N + (jnp.arange(pad, dtype=jnp.int32) % TROWS)
  src3 = jnp.concatenate([src, pad_src]).reshape(NW, NCHUNK, CH)
  dst3 = jnp.concatenate([dst, pad_dst]).reshape(NW, NCHUNK, CH)

  b02 = b0.reshape(1, FP)
  g2d = gamma.reshape(1, D)
  bt2d = beta.reshape(1, D)

  fp = _fp_contrib(x, W0, b02)

  feats = x
  for l in range(R):
    partials = _sc_aggregate(feats, src3, dst3)
    hn = _bn_layer(feats, partials, Wh[l], bh[l].reshape(1, D), g2d, bt2d)
    fp = fp + _fp_contrib(hn, Ws[l], bs[l].reshape(1, FP))
    feats = hn

  return fp.reshape(1, FP)


# speedup vs baseline: 1.0622x; 1.0349x over previous
import functools

import jax
import jax.numpy as jnp
from jax import lax
from jax.experimental import pallas as pl
from jax.experimental.pallas import tpu as pltpu
from jax.experimental.pallas import tpu_sc as plsc

N = 10000
E = 320000
D = 128
FP = 512
R = 3
EPS = 1e-5

NC = 2
NS = 16
NW = NC * NS
CH = 128
NCHUNK = 80
EPT = CH * NCHUNK
E_PAD = EPT * NW
ACC_ROWS = 10112
TROWS = ACC_ROWS - N
APT = ACC_ROWS // NS
ZROWS = 32
GRP = 16
NPAIR = NCHUNK // 2

BN_BLK = 2000
NB = N // BN_BLK


def _sc_agg_body(feats_hbm, src_hbm, dst_hbm, out_hbm,
                 srcv, dstv, rows_a, rows_b, zbuf, acc, sem_a, sem_b, sem_z):
  c = lax.axis_index("c")
  s = lax.axis_index("s")
  wid = s * NC + c

  @pl.loop(0, ZROWS)
  def _(r):
    @pl.loop(0, D, step=16)
    def _(l):
      zbuf[r, pl.ds(l, 16)] = jnp.zeros((16,), jnp.float32)

  zbase = pl.multiple_of(s * APT, 8)

  @pl.loop(0, 19)
  def _(k):
    off = pl.multiple_of(zbase + k * ZROWS, 8)
    pltpu.async_copy(zbuf, acc.at[pl.ds(off, ZROWS)], sem_z)

  pltpu.async_copy(zbuf.at[pl.ds(0, 24)],
                   acc.at[pl.ds(pl.multiple_of(zbase + 608, 8), 24)], sem_z)
  pltpu.async_copy(src_hbm.at[wid], srcv, sem_b)

  pltpu.make_async_copy(src_hbm.at[wid], srcv, sem_b).wait()
  pltpu.async_copy(feats_hbm.at[srcv.at[0]], rows_a, sem_a)

  @pl.loop(0, 19)
  def _(k):
    off = pl.multiple_of(zbase + k * ZROWS, 8)
    pltpu.make_async_copy(zbuf, acc.at[pl.ds(off, ZROWS)], sem_z).wait()

  pltpu.make_async_copy(
      zbuf.at[pl.ds(0, 24)],
      acc.at[pl.ds(pl.multiple_of(zbase + 608, 8), 24)], sem_z).wait()

  plsc.subcore_barrier()

  @pl.loop(0, NPAIR)
  def _(p):
    k0 = 2 * p
    pltpu.async_copy(feats_hbm.at[srcv.at[k0 + 1]], rows_b, sem_b)

    @pl.when(k0 % GRP == 0)
    def _():
      goff = pl.multiple_of(k0, 8)
      pltpu.sync_copy(dst_hbm.at[wid].at[pl.ds(goff, GRP)], dstv)

    r0 = k0 % GRP
    pltpu.make_async_copy(feats_hbm.at[srcv.at[k0]], rows_a, sem_a).wait()
    pltpu.sync_copy(rows_a, acc.at[dstv.at[r0]], add=True)

    @pl.when(p < NPAIR - 1)
    def _():
      pltpu.async_copy(feats_hbm.at[srcv.at[k0 + 2]], rows_a, sem_a)

    pltpu.make_async_copy(feats_hbm.at[srcv.at[k0 + 1]], rows_b, sem_b).wait()
    pltpu.sync_copy(rows_b, acc.at[dstv.at[r0 + 1]], add=True)

  plsc.subcore_barrier()

  @pl.when(s < NS - 1)
  def _():
    base = pl.multiple_of(s * 624, 8)
    pltpu.sync_copy(acc.at[pl.ds(base, 624)],
                    out_hbm.at[c].at[pl.ds(base, 624)])

  @pl.when(s == NS - 1)
  def _():
    base = (NS - 1) * 624
    pltpu.sync_copy(acc.at[pl.ds(base, 640)],
                    out_hbm.at[c].at[pl.ds(base, 640)])


@jax.jit
def _sc_aggregate(feats, src3, dst3):
  mesh = plsc.VectorSubcoreMesh(core_axis_name="c", subcore_axis_name="s")
  kern = pl.kernel(
      _sc_agg_body,
      out_type=jax.ShapeDtypeStruct((NC, N, D), jnp.float32),
      mesh=mesh,
      scratch_types=[
          pltpu.VMEM((NCHUNK, CH), jnp.int32),
          pltpu.VMEM((GRP, CH), jnp.int32),
          pltpu.VMEM((CH, D), jnp.float32),
          pltpu.VMEM((CH, D), jnp.float32),
          pltpu.VMEM((ZROWS, D), jnp.float32),
          pltpu.VMEM_SHARED((ACC_ROWS, D), jnp.float32),
          pltpu.SemaphoreType.DMA,
          pltpu.SemaphoreType.DMA,
          pltpu.SemaphoreType.DMA,
      ],
  )
  return kern(feats, src3, dst3)


def _fp_body(f_ref, w_ref, b_ref, o_ref):
  i = pl.program_id(0)
  z = lax.dot_general(f_ref[...], w_ref[...],
                      dimension_numbers=(((1,), (1,)), ((), ())),
                      preferred_element_type=jnp.float32,
                      precision=lax.Precision.DEFAULT)
  z = z + b_ref[...]
  m = jnp.max(z, axis=1, keepdims=True)
  e = jnp.exp(z - m)
  p = e / jnp.sum(e, axis=1, keepdims=True)
  blk = jnp.sum(p, axis=0, keepdims=True)

  @pl.when(i == 0)
  def _():
    o_ref[...] = jnp.zeros_like(o_ref)

  o_ref[...] += blk


@jax.jit
def _fp_contrib(feats, w, b2d):
  return pl.pallas_call(
      _fp_body,
      grid=(NB,),
      in_specs=[
          pl.BlockSpec((BN_BLK, D), lambda i: (i, 0)),
          pl.BlockSpec((FP, D), lambda i: (0, 0)),
          pl.BlockSpec((1, FP), lambda i: (0, 0)),
      ],
      out_specs=pl.BlockSpec((1, FP), lambda i: (0, 0)),
      out_shape=jax.ShapeDtypeStruct((1, FP), jnp.float32),
  )(feats, w, b2d)


def _bn_body(f_ref, p_ref, wh_ref, bh_ref, g_ref, bt_ref, hn_ref,
             h_scr, st_scr):
  i = pl.program_id(0)

  @pl.when(i < NB)
  def _():
    agg = f_ref[...] + p_ref[0] + p_ref[1]
    h = lax.dot_general(agg, wh_ref[...],
                        dimension_numbers=(((1,), (1,)), ((), ())),
                        preferred_element_type=jnp.float32,
                        precision=lax.Precision.HIGHEST)
    h = jnp.maximum(h + bh_ref[...], 0.0)
    h_scr[pl.ds(i * BN_BLK, BN_BLK), :] = h

    @pl.when(i == 0)
    def _():
      st_scr[...] = jnp.zeros_like(st_scr)

    st_scr[0:1, :] += jnp.sum(h, axis=0, keepdims=True)
    st_scr[1:2, :] += jnp.sum(h * h, axis=0, keepdims=True)

  @pl.when(i >= NB)
  def _():
    j = i - NB
    mean = st_scr[0:1, :] * (1.0 / N)
    var = st_scr[1:2, :] * (1.0 / N) - mean * mean
    rstd = lax.rsqrt(var + EPS)
    scale = g_ref[...] * rstd
    shift = bt_ref[...] - mean * scale
    h = h_scr[pl.ds(j * BN_BLK, BN_BLK), :]
    hn_ref[...] = h * scale + shift


@jax.jit
def _bn_layer(feats, partials, wh, bh2d, g2d, bt2d):
  return pl.pallas_call(
      _bn_body,
      grid=(2 * NB,),
      in_specs=[
          pl.BlockSpec((BN_BLK, D),
                       lambda i: (jnp.where(i < NB, i, NB - 1), 0)),
          pl.BlockSpec((NC, BN_BLK, D),
                       lambda i: (0, jnp.where(i < NB, i, NB - 1), 0)),
          pl.BlockSpec((D, D), lambda i: (0, 0)),
          pl.BlockSpec((1, D), lambda i: (0, 0)),
          pl.BlockSpec((1, D), lambda i: (0, 0)),
          pl.BlockSpec((1, D), lambda i: (0, 0)),
      ],
      out_specs=pl.BlockSpec((BN_BLK, D),
                             lambda i: (jnp.where(i < NB, 0, i - NB), 0)),
      out_shape=jax.ShapeDtypeStruct((N, D), jnp.float32),
      scratch_shapes=[
          pltpu.VMEM((N, D), jnp.float32),
          pltpu.VMEM((2, D), jnp.float32),
      ],
  )(feats, partials, wh, bh2d, g2d, bt2d)


def kernel(x, edge_index, W0, b0, Wh, bh, Ws, bs, gamma, beta):
  src = edge_index[0]
  dst = edge_index[1]

  pad = E_PAD - E
  pad_src = (jnp.arange(pad, dtype=jnp.int32) * 37) % N
  pad_dst = N + (jnp.arange(pad, dtype=jnp.int32) % TROWS)
  src3 = jnp.concatenate([src, pad_src]).reshape(NW, NCHUNK, CH)
  dst3 = jnp.concatenate([dst, pad_dst]).reshape(NW, NCHUNK, CH)

  b02 = b0.reshape(1, FP)
  g2d = gamma.reshape(1, D)
  bt2d = beta.reshape(1, D)

  fp = _fp_contrib(x, W0, b02)

  feats = x
  for l in range(R):
    partials = _sc_aggregate(feats, src3, dst3)
    hn = _bn_layer(feats, partials, Wh[l], bh[l].reshape(1, D), g2d, bt2d)
    fp = fp + _fp_contrib(hn, Ws[l], bs[l].reshape(1, FP))
    feats = hn

  return fp.reshape(1, FP)


# speedup vs baseline: 1.0790x; 1.0159x over previous
import functools

import jax
import jax.numpy as jnp
from jax import lax
from jax.experimental import pallas as pl
from jax.experimental.pallas import tpu as pltpu
from jax.experimental.pallas import tpu_sc as plsc

N = 10000
E = 320000
D = 128
FP = 512
R = 3
EPS = 1e-5

NC = 2
NS = 16
NW = NC * NS
CH = 128
NCHUNK = 80
EPT = CH * NCHUNK
E_PAD = EPT * NW
ACC_ROWS = 10112
TROWS = ACC_ROWS - N
APT = ACC_ROWS // NS
ZROWS = 32
GRP = 16
NPAIR = NCHUNK // 2

BN_BLK = 2000
NB = N // BN_BLK


def _sc_agg_body(feats_hbm, src_hbm, dst_hbm, out_hbm,
                 srcv, dstv, rows_a, rows_b, zbuf, acc, sem_a, sem_b, sem_z):
  c = lax.axis_index("c")
  s = lax.axis_index("s")
  wid = s * NC + c

  @pl.loop(0, ZROWS)
  def _(r):
    @pl.loop(0, D, step=16)
    def _(l):
      zbuf[r, pl.ds(l, 16)] = jnp.zeros((16,), jnp.float32)

  zbase = pl.multiple_of(s * APT, 8)

  @pl.loop(0, 19)
  def _(k):
    off = pl.multiple_of(zbase + k * ZROWS, 8)
    pltpu.async_copy(zbuf, acc.at[pl.ds(off, ZROWS)], sem_z)

  pltpu.async_copy(zbuf.at[pl.ds(0, 24)],
                   acc.at[pl.ds(pl.multiple_of(zbase + 608, 8), 24)], sem_z)
  pltpu.async_copy(src_hbm.at[wid], srcv, sem_b)

  pltpu.make_async_copy(src_hbm.at[wid], srcv, sem_b).wait()
  pltpu.async_copy(feats_hbm.at[srcv.at[0]], rows_a, sem_a)

  @pl.loop(0, 19)
  def _(k):
    off = pl.multiple_of(zbase + k * ZROWS, 8)
    pltpu.make_async_copy(zbuf, acc.at[pl.ds(off, ZROWS)], sem_z).wait()

  pltpu.make_async_copy(
      zbuf.at[pl.ds(0, 24)],
      acc.at[pl.ds(pl.multiple_of(zbase + 608, 8), 24)], sem_z).wait()

  plsc.subcore_barrier()

  @pl.loop(0, NPAIR)
  def _(p):
    k0 = 2 * p
    pltpu.async_copy(feats_hbm.at[srcv.at[k0 + 1]], rows_b, sem_b)

    @pl.when(k0 % GRP == 0)
    def _():
      goff = pl.multiple_of(k0, 8)
      pltpu.sync_copy(dst_hbm.at[wid].at[pl.ds(goff, GRP)], dstv)

    r0 = k0 % GRP
    pltpu.make_async_copy(feats_hbm.at[srcv.at[k0]], rows_a, sem_a).wait()
    pltpu.sync_copy(rows_a, acc.at[dstv.at[r0]], add=True)

    @pl.when(p < NPAIR - 1)
    def _():
      pltpu.async_copy(feats_hbm.at[srcv.at[k0 + 2]], rows_a, sem_a)

    pltpu.make_async_copy(feats_hbm.at[srcv.at[k0 + 1]], rows_b, sem_b).wait()
    pltpu.sync_copy(rows_b, acc.at[dstv.at[r0 + 1]], add=True)

  plsc.subcore_barrier()

  @pl.when(s < NS - 1)
  def _():
    base = pl.multiple_of(s * 624, 8)
    pltpu.sync_copy(acc.at[pl.ds(base, 624)],
                    out_hbm.at[c].at[pl.ds(base, 624)])

  @pl.when(s == NS - 1)
  def _():
    base = (NS - 1) * 624
    pltpu.sync_copy(acc.at[pl.ds(base, 640)],
                    out_hbm.at[c].at[pl.ds(base, 640)])


@jax.jit
def _sc_aggregate(feats, src3, dst3):
  mesh = plsc.VectorSubcoreMesh(core_axis_name="c", subcore_axis_name="s")
  kern = pl.kernel(
      _sc_agg_body,
      out_type=jax.ShapeDtypeStruct((NC, N, D), jnp.float32),
      mesh=mesh,
      scratch_types=[
          pltpu.VMEM((NCHUNK, CH), jnp.int32),
          pltpu.VMEM((GRP, CH), jnp.int32),
          pltpu.VMEM((CH, D), jnp.float32),
          pltpu.VMEM((CH, D), jnp.float32),
          pltpu.VMEM((ZROWS, D), jnp.float32),
          pltpu.VMEM_SHARED((ACC_ROWS, D), jnp.float32),
          pltpu.SemaphoreType.DMA,
          pltpu.SemaphoreType.DMA,
          pltpu.SemaphoreType.DMA,
      ],
  )
  return kern(feats, src3, dst3)


def _fp_body(f_ref, w_ref, b_ref, o_ref):
  i = pl.program_id(0)
  z = lax.dot_general(f_ref[...], w_ref[...],
                      dimension_numbers=(((1,), (1,)), ((), ())),
                      preferred_element_type=jnp.float32,
                      precision=lax.Precision.DEFAULT)
  z = z + b_ref[...]
  m = jnp.max(z, axis=1, keepdims=True)
  e = jnp.exp(z - m)
  p = e / jnp.sum(e, axis=1, keepdims=True)
  blk = jnp.sum(p, axis=0, keepdims=True)

  @pl.when(i == 0)
  def _():
    o_ref[...] = jnp.zeros_like(o_ref)

  o_ref[...] += blk


@jax.jit
def _fp_contrib(feats, w, b2d):
  return pl.pallas_call(
      _fp_body,
      grid=(NB,),
      in_specs=[
          pl.BlockSpec((BN_BLK, D), lambda i: (i, 0)),
          pl.BlockSpec((FP, D), lambda i: (0, 0)),
          pl.BlockSpec((1, FP), lambda i: (0, 0)),
      ],
      out_specs=pl.BlockSpec((1, FP), lambda i: (0, 0)),
      out_shape=jax.ShapeDtypeStruct((1, FP), jnp.float32),
  )(feats, w, b2d)


def _bn_body(f_ref, p_ref, wh_ref, bh_ref, g_ref, bt_ref, hn_ref,
             h_scr, st_scr):
  i = pl.program_id(0)

  @pl.when(i < NB)
  def _():
    agg = f_ref[...] + p_ref[0] + p_ref[1]
    h = lax.dot_general(agg, wh_ref[...],
                        dimension_numbers=(((1,), (1,)), ((), ())),
                        preferred_element_type=jnp.float32,
                        precision=lax.Precision.DEFAULT)
    h = jnp.maximum(h + bh_ref[...], 0.0)
    h_scr[pl.ds(i * BN_BLK, BN_BLK), :] = h

    @pl.when(i == 0)
    def _():
      st_scr[...] = jnp.zeros_like(st_scr)

    st_scr[0:1, :] += jnp.sum(h, axis=0, keepdims=True)
    st_scr[1:2, :] += jnp.sum(h * h, axis=0, keepdims=True)

  @pl.when(i >= NB)
  def _():
    j = i - NB
    mean = st_scr[0:1, :] * (1.0 / N)
    var = st_scr[1:2, :] * (1.0 / N) - mean * mean
    rstd = lax.rsqrt(var + EPS)
    scale = g_ref[...] * rstd
    shift = bt_ref[...] - mean * scale
    h = h_scr[pl.ds(j * BN_BLK, BN_BLK), :]
    hn_ref[...] = h * scale + shift


@jax.jit
def _bn_layer(feats, partials, wh, bh2d, g2d, bt2d):
  return pl.pallas_call(
      _bn_body,
      grid=(2 * NB,),
      in_specs=[
          pl.BlockSpec((BN_BLK, D),
                       lambda i: (jnp.where(i < NB, i, NB - 1), 0)),
          pl.BlockSpec((NC, BN_BLK, D),
                       lambda i: (0, jnp.where(i < NB, i, NB - 1), 0)),
          pl.BlockSpec((D, D), lambda i: (0, 0)),
          pl.BlockSpec((1, D), lambda i: (0, 0)),
          pl.BlockSpec((1, D), lambda i: (0, 0)),
          pl.BlockSpec((1, D), lambda i: (0, 0)),
      ],
      out_specs=pl.BlockSpec((BN_BLK, D),
                             lambda i: (jnp.where(i < NB, 0, i - NB), 0)),
      out_shape=jax.ShapeDtypeStruct((N, D), jnp.float32),
      scratch_shapes=[
          pltpu.VMEM((N, D), jnp.float32),
          pltpu.VMEM((2, D), jnp.float32),
      ],
  )(feats, partials, wh, bh2d, g2d, bt2d)


def kernel(x, edge_index, W0, b0, Wh, bh, Ws, bs, gamma, beta):
  src = edge_index[0]
  dst = edge_index[1]

  pad = E_PAD - E
  pad_src = (jnp.arange(pad, dtype=jnp.int32) * 37) % N
  pad_dst = N + (jnp.arange(pad, dtype=jnp.int32) % TROWS)
  src3 = jnp.concatenate([src, pad_src]).reshape(NW, NCHUNK, CH)
  dst3 = jnp.concatenate([dst, pad_dst]).reshape(NW, NCHUNK, CH)

  b02 = b0.reshape(1, FP)
  g2d = gamma.reshape(1, D)
  bt2d = beta.reshape(1, D)

  fp = _fp_contrib(x, W0, b02)

  feats = x
  for l in range(R):
    partials = _sc_aggregate(feats, src3, dst3)
    hn = _bn_layer(feats, partials, Wh[l], bh[l].reshape(1, D), g2d, bt2d)
    fp = fp + _fp_contrib(hn, Ws[l], bs[l].reshape(1, FP))
    feats = hn

  return fp.reshape(1, FP)
